# baseline (speedup 1.0000x reference)
import functools

import jax
import jax.numpy as jnp
from jax import lax
from jax.experimental import pallas as pl
from jax.experimental.pallas import tpu as pltpu
from jax.experimental.pallas import tpu_sc as plsc

_B = 16384
_D = 256
_NC = 2
_NS = 16
_NW = _NC * _NS
_CH = 64

_FT_CLAMP = 127.0 / 127.0
_HL_CLAMP = 127.0 / 64.0
_LEAK = 0.01
_SIGMOID_SCALE = 400.0

_NSPLIT = 2
_BS = _B // _NSPLIT

_BT = 2048

_NBUF = 6


def _sc_gather(table, wb_idx, nb):
    mesh = plsc.VectorSubcoreMesh(core_axis_name="c", subcore_axis_name="s")
    bpw = nb // _NW
    nch = bpw // _CH
    nchunks = 2 * nch

    @functools.partial(
        pl.kernel,
        mesh=mesh,
        out_type=(
            jax.ShapeDtypeStruct((nb, _D), jnp.float32),
            jax.ShapeDtypeStruct((nb, _D), jnp.float32),
        ),
        scratch_types=(
            [pltpu.VMEM((2, bpw), jnp.int32)]
            + [pltpu.VMEM((_CH, _D), jnp.float32) for _ in range(_NBUF)]
            + [pltpu.SemaphoreType.DMA for _ in range(2 * _NBUF)]
        ),
    )
    def gather_kernel(table_hbm, wb_idx_hbm, w_out, b_out,
                      idx_v, *bufs_and_sems):
        rows = bufs_and_sems[:_NBUF]
        gsem = bufs_and_sems[_NBUF:2 * _NBUF]
        wsem = bufs_and_sems[2 * _NBUF:]
        wid = lax.axis_index("s") * _NC + lax.axis_index("c")
        base = wid * bpw

        pltpu.sync_copy(wb_idx_hbm.at[:, pl.ds(base, bpw)], idx_v)

        def gather_start(k, b):
            side = 0 if k < nch else 1
            idx_slice = idx_v.at[side, pl.ds((k % nch) * _CH, _CH)]
            return pltpu.async_copy(table_hbm.at[idx_slice], rows[b], gsem[b])

        def write_start(k, b):
            out_hbm = w_out if k < nch else b_out
            off = base + (k % nch) * _CH
            return pltpu.async_copy(rows[b], out_hbm.at[pl.ds(off, _CH)],
                                    wsem[b])

        gathers = [None] * nchunks
        writes = [None] * nchunks
        for k in range(min(_NBUF, nchunks)):
            gathers[k] = gather_start(k, k % _NBUF)
        for k in range(nchunks):
            b = k % _NBUF
            gathers[k].wait()
            writes[k] = write_start(k, b)
            nxt = k + _NBUF
            if nxt < nchunks:
                writes[k].wait()
                gathers[nxt] = gather_start(nxt, b)
        for k in range(max(0, nchunks - _NBUF), nchunks):
            writes[k].wait()

    return gather_kernel(table, wb_idx)


def _clipped_relu(x, upper):
    return _LEAK * x + (1.0 - _LEAK) * jnp.clip(x, 0.0, upper)


def _mlp_body(w_ref, b_ref, stm_ref, ftb_ref, l1a_ref, l1b_ref, l1bias_ref,
              l2_ref, l2bias_ref, ow_ref, obias_ref, o_ref):
    ftb = ftb_ref[...]
    wf = _clipped_relu(w_ref[...] + ftb, _FT_CLAMP)
    bf = _clipped_relu(b_ref[...] + ftb, _FT_CLAMP)
    l1a = l1a_ref[...]
    l1b = l1b_ref[...]
    l1bias = l1bias_ref[...]
    dot = lambda a, b: jnp.dot(a, b, preferred_element_type=jnp.float32)
    h_white = dot(wf, l1a) + dot(bf, l1b) + l1bias
    h_black = dot(bf, l1a) + dot(wf, l1b) + l1bias
    l2t = l2_ref[...]
    l2bias = l2bias_ref[...]
    xw = _clipped_relu(h_white, _HL_CLAMP)
    xb = _clipped_relu(h_black, _HL_CLAMP)
    xw = _clipped_relu(dot(xw, l2t) + l2bias, _HL_CLAMP)
    xb = _clipped_relu(dot(xb, l2t) + l2bias, _HL_CLAMP)
    ow_row = ow_ref[...]
    obias = obias_ref[0, 0]
    rdot = lambda a, b: lax.dot_general(
        a, b, (((1,), (1,)), ((), ())), preferred_element_type=jnp.float32)
    o_white = rdot(ow_row, xw) + obias
    o_black = rdot(ow_row, xb) + obias
    is_white = stm_ref[...] == 0
    o_ref[...] = jnp.where(is_white, o_white, -o_black) * _SIGMOID_SCALE


def _tc_mlp(w_rows, b_rows, stm1, ft_bias2, l1a, l1b, l1_bias2, l2t, l2_bias2,
            ow_row, out_bias2, nb):
    grid = (nb // _BT,)
    full = lambda shape: pl.BlockSpec(shape, lambda i: (0, 0))
    return pl.pallas_call(
        _mlp_body,
        grid=grid,
        in_specs=[
            pl.BlockSpec((_BT, _D), lambda i: (i, 0)),
            pl.BlockSpec((_BT, _D), lambda i: (i, 0)),
            pl.BlockSpec((1, _BT), lambda i: (0, i)),
            full((1, _D)),
            full((_D, 32)),
            full((_D, 32)),
            full((1, 32)),
            full((32, 32)),
            full((1, 32)),
            full((1, 32)),
            full((1, 1)),
        ],
        out_specs=pl.BlockSpec((1, _BT), lambda i: (0, i)),
        out_shape=jax.ShapeDtypeStruct((1, nb), jnp.float32),
    )(w_rows, b_rows, stm1, ft_bias2, l1a, l1b, l1_bias2, l2t, l2_bias2,
      ow_row, out_bias2)


def kernel(w_idx, w_off, b_idx, b_off, stm, ft_weight, ft_bias,
           l1_w, l1_b, l2_w, l2_b, out_w, out_b):
    l1t = l1_w.T
    ftb2 = ft_bias.reshape(1, _D)
    l1a, l1b2 = l1t[:_D], l1t[_D:]
    l1bias2 = l1_b.reshape(1, 32)
    l2t = l2_w.T
    l2bias2 = l2_b.reshape(1, 32)
    ow_row = out_w.reshape(1, 32)
    obias2 = out_b.reshape(1, 1)
    stm_row = stm.reshape(1, _B)
    wb_idx = jnp.stack([w_idx, b_idx])

    outs = []
    for s in range(_NSPLIT):
        lo = s * _BS
        w_rows, b_rows = _sc_gather(
            ft_weight, lax.slice(wb_idx, (0, lo), (2, lo + _BS)), _BS)
        outs.append(_tc_mlp(
            w_rows, b_rows, lax.slice(stm_row, (0, lo), (1, lo + _BS)),
            ftb2, l1a, l1b2, l1bias2, l2t, l2bias2, ow_row, obias2, _BS))
    return jnp.concatenate(outs, axis=1).reshape(_B, 1)


# speedup vs baseline: 1.0102x; 1.0102x over previous
import functools

import jax
import jax.numpy as jnp
from jax import lax
from jax.experimental import pallas as pl
from jax.experimental.pallas import tpu as pltpu
from jax.experimental.pallas import tpu_sc as plsc

_B = 16384
_D = 256
_NC = 2
_NS = 16
_NW = _NC * _NS
_CH = 64

_FT_CLAMP = 127.0 / 127.0
_HL_CLAMP = 127.0 / 64.0
_LEAK = 0.01
_SIGMOID_SCALE = 400.0

_NSPLIT = 2
_BS = _B // _NSPLIT

_BT = 4096

_NBUF = 6


def _sc_gather(table, wb_idx, nb):
    mesh = plsc.VectorSubcoreMesh(core_axis_name="c", subcore_axis_name="s")
    bpw = nb // _NW
    nch = bpw // _CH
    nchunks = 2 * nch

    @functools.partial(
        pl.kernel,
        mesh=mesh,
        out_type=(
            jax.ShapeDtypeStruct((nb, _D), jnp.float32),
            jax.ShapeDtypeStruct((nb, _D), jnp.float32),
        ),
        scratch_types=(
            [pltpu.VMEM((2, bpw), jnp.int32)]
            + [pltpu.VMEM((_CH, _D), jnp.float32) for _ in range(_NBUF)]
            + [pltpu.SemaphoreType.DMA for _ in range(2 * _NBUF)]
        ),
    )
    def gather_kernel(table_hbm, wb_idx_hbm, w_out, b_out,
                      idx_v, *bufs_and_sems):
        rows = bufs_and_sems[:_NBUF]
        gsem = bufs_and_sems[_NBUF:2 * _NBUF]
        wsem = bufs_and_sems[2 * _NBUF:]
        wid = lax.axis_index("s") * _NC + lax.axis_index("c")
        base = wid * bpw

        pltpu.sync_copy(wb_idx_hbm.at[:, pl.ds(base, bpw)], idx_v)

        def gather_start(k, b):
            side = 0 if k < nch else 1
            idx_slice = idx_v.at[side, pl.ds((k % nch) * _CH, _CH)]
            return pltpu.async_copy(table_hbm.at[idx_slice], rows[b], gsem[b])

        def write_start(k, b):
            out_hbm = w_out if k < nch else b_out
            off = base + (k % nch) * _CH
            return pltpu.async_copy(rows[b], out_hbm.at[pl.ds(off, _CH)],
                                    wsem[b])

        gathers = [None] * nchunks
        writes = [None] * nchunks
        for k in range(min(_NBUF, nchunks)):
            gathers[k] = gather_start(k, k % _NBUF)
        for k in range(nchunks):
            b = k % _NBUF
            gathers[k].wait()
            writes[k] = write_start(k, b)
            nxt = k + _NBUF
            if nxt < nchunks:
                writes[k].wait()
                gathers[nxt] = gather_start(nxt, b)
        for k in range(max(0, nchunks - _NBUF), nchunks):
            writes[k].wait()

    return gather_kernel(table, wb_idx)


def _clipped_relu(x, upper):
    return _LEAK * x + (1.0 - _LEAK) * jnp.clip(x, 0.0, upper)


def _mlp_body(w_ref, b_ref, stm_ref, ftb_ref, l1a_ref, l1b_ref, l1bias_ref,
              l2_ref, l2bias_ref, ow_ref, obias_ref, o_ref):
    ftb = ftb_ref[...]
    wf = _clipped_relu(w_ref[...] + ftb, _FT_CLAMP)
    bf = _clipped_relu(b_ref[...] + ftb, _FT_CLAMP)
    l1a = l1a_ref[...]
    l1b = l1b_ref[...]
    l1bias = l1bias_ref[...]
    dot = lambda a, b: jnp.dot(a, b, preferred_element_type=jnp.float32)
    h_white = dot(wf, l1a) + dot(bf, l1b) + l1bias
    h_black = dot(bf, l1a) + dot(wf, l1b) + l1bias
    l2t = l2_ref[...]
    l2bias = l2bias_ref[...]
    xw = _clipped_relu(h_white, _HL_CLAMP)
    xb = _clipped_relu(h_black, _HL_CLAMP)
    xw = _clipped_relu(dot(xw, l2t) + l2bias, _HL_CLAMP)
    xb = _clipped_relu(dot(xb, l2t) + l2bias, _HL_CLAMP)
    ow_row = ow_ref[...]
    obias = obias_ref[0, 0]
    rdot = lambda a, b: lax.dot_general(
        a, b, (((1,), (1,)), ((), ())), preferred_element_type=jnp.float32)
    o_white = rdot(ow_row, xw) + obias
    o_black = rdot(ow_row, xb) + obias
    is_white = stm_ref[...] == 0
    o_ref[...] = jnp.where(is_white, o_white, -o_black) * _SIGMOID_SCALE


def _tc_mlp(w_rows, b_rows, stm1, ft_bias2, l1a, l1b, l1_bias2, l2t, l2_bias2,
            ow_row, out_bias2, nb):
    grid = (nb // _BT,)
    full = lambda shape: pl.BlockSpec(shape, lambda i: (0, 0))
    return pl.pallas_call(
        _mlp_body,
        grid=grid,
        in_specs=[
            pl.BlockSpec((_BT, _D), lambda i: (i, 0)),
            pl.BlockSpec((_BT, _D), lambda i: (i, 0)),
            pl.BlockSpec((1, _BT), lambda i: (0, i)),
            full((1, _D)),
            full((_D, 32)),
            full((_D, 32)),
            full((1, 32)),
            full((32, 32)),
            full((1, 32)),
            full((1, 32)),
            full((1, 1)),
        ],
        out_specs=pl.BlockSpec((1, _BT), lambda i: (0, i)),
        out_shape=jax.ShapeDtypeStruct((1, nb), jnp.float32),
    )(w_rows, b_rows, stm1, ft_bias2, l1a, l1b, l1_bias2, l2t, l2_bias2,
      ow_row, out_bias2)


def kernel(w_idx, w_off, b_idx, b_off, stm, ft_weight, ft_bias,
           l1_w, l1_b, l2_w, l2_b, out_w, out_b):
    l1t = l1_w.T
    ftb2 = ft_bias.reshape(1, _D)
    l1a, l1b2 = l1t[:_D], l1t[_D:]
    l1bias2 = l1_b.reshape(1, 32)
    l2t = l2_w.T
    l2bias2 = l2_b.reshape(1, 32)
    ow_row = out_w.reshape(1, 32)
    obias2 = out_b.reshape(1, 1)
    stm_row = stm.reshape(1, _B)
    wb_idx = jnp.stack([w_idx, b_idx])

    outs = []
    for s in range(_NSPLIT):
        lo = s * _BS
        w_rows, b_rows = _sc_gather(
            ft_weight, lax.slice(wb_idx, (0, lo), (2, lo + _BS)), _BS)
        outs.append(_tc_mlp(
            w_rows, b_rows, lax.slice(stm_row, (0, lo), (1, lo + _BS)),
            ftb2, l1a, l1b2, l1bias2, l2t, l2bias2, ow_row, obias2, _BS))
    return jnp.concatenate(outs, axis=1).reshape(_B, 1)
